# baseline (speedup 1.0000x reference)
import functools

import jax
import jax.numpy as jnp
import numpy as np
from jax import lax
from jax.experimental import pallas as pl
from jax.experimental.pallas import tpu as pltpu
from jax.experimental.pallas import tpu_sc as plsc


def _tap_tables_tc(e_pad, conv_wt, conv_b2):
    def body(e_ref, w_ref, b_ref, out_ref):
        e = e_ref[...]
        for k in range(3):
            wk = w_ref[k]
            tk = lax.dot_general(
                e, wk,
                dimension_numbers=(((1,), (1,)), ((), ())),
                preferred_element_type=jnp.float32,
            )
            if k == 1:
                tk = tk + b_ref[...]
            out_ref[:, 32 * k:32 * (k + 1)] = tk
        out_ref[:, 96:128] = jnp.zeros((256, 32), jnp.float32)
        return None

    return pl.pallas_call(
        body,
        out_shape=jax.ShapeDtypeStruct((256, 128), jnp.float32),
    )(e_pad, conv_wt, conv_b2)


def _make_sc_kernel(n_tok):
    n_workers = 32
    tok_w = n_tok // n_workers
    chunk = 4
    n_chunks = tok_w // chunk
    rows_c = chunk * 22
    ids_w = tok_w * 22
    out_w = tok_w * 32

    mesh = plsc.VectorSubcoreMesh(core_axis_name="c", subcore_axis_name="s")

    @functools.partial(
        pl.kernel,
        out_type=jax.ShapeDtypeStruct((n_tok * 32,), jnp.float32),
        mesh=mesh,
        scratch_types=[
            pltpu.VMEM((ids_w,), jnp.int32),
            pltpu.VMEM((rows_c, 128), jnp.float32),
            pltpu.VMEM((rows_c, 128), jnp.float32),
            pltpu.VMEM((660,), jnp.float32),
            pltpu.VMEM((660,), jnp.float32),
            pltpu.VMEM((660,), jnp.float32),
            pltpu.VMEM((660,), jnp.float32),
            pltpu.VMEM((out_w,), jnp.float32),
            pltpu.VMEM((640,), jnp.int32),
            pltpu.SemaphoreType.DMA,
            pltpu.SemaphoreType.DMA,
        ],
        compiler_params=pltpu.CompilerParams(needs_layout_passes=False),
    )
    def sc_main(ids_hbm, tab_hbm, pat_hbm, out_hbm,
                ids_v, rows0, rows1, y0, y1, y2, y3, out_v, pat_v, g0, g1):
        wid = lax.axis_index("s") * 2 + lax.axis_index("c")
        pltpu.sync_copy(ids_hbm.at[pl.ds(wid * ids_w, ids_w)], ids_v)
        pltpu.sync_copy(pat_hbm, pat_v)
        ybufs = (y0, y1, y2, y3)

        def fire(ch, rows_b, sem):
            pltpu.async_copy(
                tab_hbm.at[ids_v.at[pl.ds(ch * rows_c, rows_c)]],
                rows_b,
                sem,
            )

        def drain(rows_b, sem):
            pltpu.make_async_copy(
                tab_hbm.at[pl.ds(0, rows_c)],
                rows_b,
                sem,
            ).wait()

        def compute(ch, rows_b):
            for t in range(chunk):
                yb = ybufs[t % 4]
                rb = t * 22
                for w in range(20):
                    for h in range(2):
                        v = (
                            rows_b[rb + w, pl.ds(16 * h, 16)]
                            + rows_b[rb + w + 1, pl.ds(32 + 16 * h, 16)]
                            + rows_b[rb + w + 2, pl.ds(64 + 16 * h, 16)]
                        )
                        yb[pl.ds(33 * w + 16 * h, 16)] = v
                base = (ch * chunk + t) * 32
                for jh in range(2):
                    acc = None
                    for i in range(20):
                        pat = pat_v[pl.ds(32 * i + 16 * jh, 16)]
                        vals = plsc.load_gather(yb, [pat])
                        acc = vals if acc is None else jnp.maximum(acc, vals)
                    out_v[pl.ds(base + 16 * jh, 16)] = acc

        fire(0, rows0, g0)
        fire(1, rows1, g1)

        def pair(p, carry):
            ch0 = 2 * p
            drain(rows0, g0)
            compute(ch0, rows0)

            @pl.when(ch0 + 2 < n_chunks)
            def _():
                fire(ch0 + 2, rows0, g0)

            drain(rows1, g1)
            compute(ch0 + 1, rows1)

            @pl.when(ch0 + 3 < n_chunks)
            def _():
                fire(ch0 + 3, rows1, g1)

            return carry

        lax.fori_loop(0, n_chunks // 2, pair, 0)
        pltpu.sync_copy(out_v, out_hbm.at[pl.ds(wid * out_w, out_w)])

    return sc_main


def kernel(char_ids, embed_table, conv_w, conv_b):
    b, s, w = char_ids.shape
    d = embed_table.shape[1]
    n_tok = b * s

    e_pad = jnp.pad(embed_table.astype(jnp.float32), ((0, 128), (0, 0)))
    conv_wt = conv_w.astype(jnp.float32).transpose(2, 0, 1)
    conv_b2 = conv_b.astype(jnp.float32).reshape(1, d)

    tab = _tap_tables_tc(e_pad, conv_wt, conv_b2)

    ids = char_ids.astype(jnp.int32).reshape(n_tok, w)
    ids_pad = jnp.pad(ids, ((0, 0), (1, 1)), constant_values=128)
    ids_flat = ids_pad.reshape(n_tok * 22)

    f = np.arange(640)
    pat = jnp.asarray(33 * (f % 20) + f // 20, dtype=jnp.int32)

    sc_main = _make_sc_kernel(n_tok)
    out_flat = sc_main(ids_flat, tab, pat)
    return out_flat.reshape(b, s, d)


# speedup vs baseline: 5.7890x; 5.7890x over previous
import functools

import jax
import jax.numpy as jnp
from jax import lax
from jax.experimental import pallas as pl
from jax.experimental.pallas import tpu as pltpu
from jax.experimental.pallas import tpu_sc as plsc

_REP = 8
_RSTRIDE = 2 * 128 * 32 + 2
_T1BASE = _REP * _RSTRIDE


def _tap_tables_tc(embed, conv_wt, conv_b2):
    def body(e_ref, w_ref, b_ref, out_ref):
        e = e_ref[...]
        for k in range(3):
            wk = w_ref[k]
            tk = lax.dot_general(
                e, wk,
                dimension_numbers=(((1,), (1,)), ((), ())),
                preferred_element_type=jnp.float32,
            )
            if k == 1:
                tk = tk + b_ref[...]
            out_ref[k] = tk
        return None

    return pl.pallas_call(
        body,
        out_shape=jax.ShapeDtypeStruct((3, 128, 32), jnp.float32),
    )(embed, conv_wt, conv_b2)


def _make_sc_kernel(n_tok):
    n_workers = 32
    tok_w = n_tok // n_workers
    groups = tok_w // 16
    ids_w = tok_w * 20
    tab_w = _T1BASE + 128 * 33
    out_w = tok_w * 33

    mesh = plsc.VectorSubcoreMesh(core_axis_name="c", subcore_axis_name="s")

    @functools.partial(
        pl.kernel,
        out_type=jax.ShapeDtypeStruct((n_tok * 33,), jnp.float32),
        mesh=mesh,
        scratch_types=[
            pltpu.VMEM((ids_w,), jnp.int32),
            pltpu.VMEM((tab_w,), jnp.float32),
            pltpu.VMEM((16 * 33,), jnp.float32),
            pltpu.SemaphoreType.DMA,
        ],
        compiler_params=pltpu.CompilerParams(needs_layout_passes=False),
    )
    def sc_main(ids_hbm, tab_hbm, out_hbm, ids_v, tab_v, outc_v, osem):
        wid = lax.axis_index("s") * 2 + lax.axis_index("c")
        pltpu.sync_copy(ids_hbm.at[pl.ds(wid * ids_w, ids_w)], ids_v)
        pltpu.sync_copy(tab_hbm, tab_v)

        iota = lax.iota(jnp.int32, 16)
        iota20 = iota * 20
        iota33 = iota * 33
        rbase = (iota & 7) * _RSTRIDE

        def group(g, carry):
            cbase = g * (16 * 20)
            craw = []
            c32 = []
            for e in range(20):
                c = plsc.load_gather(ids_v, [iota20 + (cbase + e)])
                craw.append(c)
                c32.append(c * 32 + rbase)
            @pl.when(g > 0)
            def _():
                pltpu.make_async_copy(
                    outc_v, out_hbm.at[pl.ds(0, 16 * 33)], osem,
                ).wait()
            for j in range(32):
                acc = None
                for i in range(20):
                    f = 32 * i + j
                    w = f % 20
                    o = f // 20
                    v = plsc.load_gather(
                        tab_v, [craw[w] * 33 + (_T1BASE + o)])
                    if w > 0:
                        v = v + plsc.load_gather(tab_v, [c32[w - 1] + o])
                    if w < 19:
                        v = v + plsc.load_gather(
                            tab_v, [c32[w + 1] + (4096 + o)])
                    acc = v if acc is None else jnp.maximum(acc, v)
                plsc.store_scatter(outc_v, [iota33 + j], acc)
            pltpu.async_copy(
                outc_v, out_hbm.at[pl.ds(wid * out_w + g * (16 * 33),
                                         16 * 33)], osem)
            return carry

        lax.fori_loop(0, groups, group, 0)
        pltpu.make_async_copy(
            outc_v, out_hbm.at[pl.ds(0, 16 * 33)], osem,
        ).wait()

    return sc_main


def kernel(char_ids, embed_table, conv_w, conv_b):
    b, s, w = char_ids.shape
    d = embed_table.shape[1]
    n_tok = b * s

    conv_wt = conv_w.astype(jnp.float32).transpose(2, 0, 1)
    conv_b2 = conv_b.astype(jnp.float32).reshape(1, d)

    tab = _tap_tables_tc(embed_table.astype(jnp.float32), conv_wt, conv_b2)
    t02 = jnp.concatenate([tab[0].reshape(-1), tab[2].reshape(-1),
                           jnp.zeros((2,), jnp.float32)])
    t1p = jnp.pad(tab[1], ((0, 0), (0, 1))).reshape(-1)
    tab_rep = jnp.concatenate([jnp.tile(t02, _REP), t1p])

    ids_flat = char_ids.astype(jnp.int32).reshape(n_tok * w)

    sc_main = _make_sc_kernel(n_tok)
    out_flat = sc_main(ids_flat, tab_rep)
    return out_flat.reshape(n_tok, 33)[:, :d].reshape(b, s, d)
